# Initial kernel scaffold; baseline (speedup 1.0000x reference)
import jax
import jax.numpy as jnp
from jax.experimental import pallas as pl


def kernel(question_ids, responses, question_table, interaction_table, ln_gamma, ln_beta):
    raise NotImplementedError("write your pallas kernel here")


# speedup vs baseline: 1.1959x; 1.1959x over previous
import functools

import jax
import jax.numpy as jnp
from jax import lax
from jax.experimental import pallas as pl
from jax.experimental.pallas import tpu as pltpu
from jax.experimental.pallas import tpu_sc as plsc

_NUM_Q = 100000
_D = 64
_B, _T = 1024, 200
_N = _B * _T
_NC, _NS = 2, 16
_NW = _NC * _NS
_NT = _N // _NW
_SUB = 128
_ROWS = _NT // _SUB
_NSUB = 5
_C = _SUB * _NSUB
_G = _NT // _C
_MAX_IID = 2 * _NUM_Q
_EPS = 1e-5


def _tec_body(qid_hbm, resp_hbm, qtab_hbm, itab_hbm, gam_hbm, bet_hbm,
              x_hbm, qemb_hbm,
              qspan_v, ispan_v, qrows_v, irows_v, gam_v, bet_v, sem):
    cid = lax.axis_index("c")
    sid = lax.axis_index("s")
    wid = sid * _NC + cid
    pltpu.sync_copy(gam_hbm, gam_v)
    pltpu.sync_copy(bet_hbm, bet_v)
    pltpu.sync_copy(qid_hbm.at[wid], qspan_v)
    pltpu.sync_copy(resp_hbm.at[wid], ispan_v)

    def idx_row(r, c0):
        for t in range(_SUB // 16):
            s = pl.ds(t * 16, 16)
            q = qspan_v[r, s]
            rr = ispan_v[r, s]
            rr = jnp.minimum(jnp.maximum(rr, 0), 1)
            iid = q + q + rr
            iid = jnp.where(q == 0, 0, iid)
            ispan_v[r, s] = jnp.minimum(iid, _MAX_IID)
        return c0

    lax.fori_loop(0, _ROWS, idx_row, 0)

    def chunk(g, carry):
        tok0 = wid * _NT + g * _C
        copies = []
        for j in range(_NSUB):
            copies.append(pltpu.async_copy(
                qtab_hbm.at[qspan_v.at[g * _NSUB + j]],
                qrows_v.at[pl.ds(j * _SUB, _SUB)], sem))
            copies.append(pltpu.async_copy(
                itab_hbm.at[ispan_v.at[g * _NSUB + j]],
                irows_v.at[pl.ds(j * _SUB, _SUB)], sem))
        for cp in copies:
            cp.wait()

        inv_d = jnp.float32(1.0 / _D)
        lanes = lax.iota(jnp.int32, 16)

        def hsum(v):
            for k in (8, 4, 2, 1):
                v = v + v.at[lanes ^ k].get(mode="promise_in_bounds")
            return v

        def tok(t, c2):
            v = []
            for j in range(_D // 16):
                s = pl.ds(j * 16, 16)
                v.append(qrows_v[t, s] + irows_v[t, s])
            stot = (v[0] + v[1]) + (v[2] + v[3])
            sstot = (v[0] * v[0] + v[1] * v[1]) + (v[2] * v[2] + v[3] * v[3])
            mv = hsum(stot) * inv_d
            vv = hsum(sstot) * inv_d - mv * mv + jnp.float32(_EPS)
            iy = jnp.int32(0x5F3759DF) - lax.shift_right_logical(
                plsc.bitcast(vv, jnp.int32), 1)
            y = plsc.bitcast(iy, jnp.float32)
            for _ in range(3):
                y = y * (jnp.float32(1.5) - jnp.float32(0.5) * vv * y * y)
            nb = mv * y
            for j in range(_D // 16):
                s = pl.ds(j * 16, 16)
                irows_v[t, s] = (v[j] * y - nb) * gam_v[s] + bet_v[s]
            return c2

        lax.fori_loop(0, _C, tok, 0)
        pltpu.sync_copy(qrows_v, qemb_hbm.at[pl.ds(tok0, _C)])
        pltpu.sync_copy(irows_v, x_hbm.at[pl.ds(tok0, _C)])
        return carry

    lax.fori_loop(0, _G, chunk, 0)


def kernel(question_ids, responses, question_table, interaction_table,
           ln_gamma, ln_beta):
    qid = question_ids.reshape(_NW, _ROWS, _SUB).astype(jnp.int32)
    resp = responses.reshape(_NW, _ROWS, _SUB).astype(jnp.int32)
    mesh = plsc.VectorSubcoreMesh(core_axis_name="c", subcore_axis_name="s")
    run = pl.kernel(
        _tec_body,
        out_type=(
            jax.ShapeDtypeStruct((_N, _D), jnp.float32),
            jax.ShapeDtypeStruct((_N, _D), jnp.float32),
        ),
        mesh=mesh,
        compiler_params=pltpu.CompilerParams(
            needs_layout_passes=False, use_tc_tiling_on_sc=False),
        scratch_types=[
            pltpu.VMEM((_ROWS, _SUB), jnp.int32),
            pltpu.VMEM((_ROWS, _SUB), jnp.int32),
            pltpu.VMEM((_C, _D), jnp.float32),
            pltpu.VMEM((_C, _D), jnp.float32),
            pltpu.VMEM((_D,), jnp.float32),
            pltpu.VMEM((_D,), jnp.float32),
            pltpu.SemaphoreType.DMA,
        ],
    )
    x, qemb = run(qid, resp, question_table, interaction_table,
                  ln_gamma, ln_beta)
    return (x.reshape(_B, _T, _D), qemb.reshape(_B, _T, _D))


# speedup vs baseline: 1.4484x; 1.2111x over previous
import functools

import jax
import jax.numpy as jnp
from jax import lax
from jax.experimental import pallas as pl
from jax.experimental.pallas import tpu as pltpu
from jax.experimental.pallas import tpu_sc as plsc

_NUM_Q = 100000
_D = 64
_B, _T = 1024, 200
_N = _B * _T
_NC, _NS = 2, 16
_NW = _NC * _NS
_NT = _N // _NW
_C = 320
_G = _NT // _C
_MAX_IID = 2 * _NUM_Q
_EPS = 1e-5


def _tec_body(qid_hbm, resp_hbm, qtab_hbm, itab_hbm, gam_hbm, bet_hbm,
              x_hbm, qemb_hbm,
              qspan_v, ispan_v, qr0, ir0, qr1, ir1, gam_v, bet_v,
              g0, g1, wq0, wq1):
    cid = lax.axis_index("c")
    sid = lax.axis_index("s")
    wid = sid * _NC + cid
    span0 = pl.multiple_of(wid * _NT, _NT)
    pltpu.sync_copy(gam_hbm, gam_v)
    pltpu.sync_copy(bet_hbm, bet_v)
    pltpu.sync_copy(qid_hbm.at[pl.ds(span0, _NT)], qspan_v)
    pltpu.sync_copy(resp_hbm.at[pl.ds(span0, _NT)], ispan_v)

    def idx_row(r, c0):
        s = pl.ds(r * 16, 16)
        q = qspan_v[s]
        rr = ispan_v[s]
        rr = jnp.minimum(jnp.maximum(rr, 0), 1)
        iid = q + q + rr
        iid = jnp.where(q == 0, 0, iid)
        ispan_v[s] = jnp.minimum(iid, _MAX_IID)
        return c0

    lax.fori_loop(0, _NT // 16, idx_row, 0)

    bufs = ((qr0, ir0, g0, wq0), (qr1, ir1, g1, wq1))

    def issue_gathers(c, qr, ir, sem):
        off = c * _C
        pltpu.async_copy(qtab_hbm.at[qspan_v.at[pl.ds(off, _C)]], qr, sem)
        pltpu.async_copy(itab_hbm.at[ispan_v.at[pl.ds(off, _C)]], ir, sem)

    def drain_gathers(qr, ir, sem):
        pltpu.make_async_copy(qtab_hbm.at[qspan_v.at[pl.ds(0, _C)]], qr,
                              sem).wait()
        pltpu.make_async_copy(itab_hbm.at[ispan_v.at[pl.ds(0, _C)]], ir,
                              sem).wait()

    inv_d = jnp.float32(1.0 / _D)
    lanes = lax.iota(jnp.int32, 16)
    gams = [gam_v[pl.ds(j * 16, 16)] for j in range(_D // 16)]
    bets = [bet_v[pl.ds(j * 16, 16)] for j in range(_D // 16)]

    def hsum(v):
        for k in (8, 4, 2, 1):
            v = v + v.at[lanes ^ k].get(mode="promise_in_bounds")
        return v

    def make_tok(qr, ir):
        def tok(t, c2):
            v = []
            for j in range(_D // 16):
                s = pl.ds(j * 16, 16)
                v.append(qr[t, s] + ir[t, s])
            stot = (v[0] + v[1]) + (v[2] + v[3])
            sstot = (v[0] * v[0] + v[1] * v[1]) + (v[2] * v[2] + v[3] * v[3])
            mv = hsum(stot) * inv_d
            vv = hsum(sstot) * inv_d - mv * mv + jnp.float32(_EPS)
            iy = jnp.int32(0x5F3759DF) - lax.shift_right_logical(
                plsc.bitcast(vv, jnp.int32), 1)
            y = plsc.bitcast(iy, jnp.float32)
            for _ in range(3):
                y = y * (jnp.float32(1.5) - jnp.float32(0.5) * vv * y * y)
            nb = mv * y
            for j in range(_D // 16):
                s = pl.ds(j * 16, 16)
                ir[t, s] = (v[j] * y - nb) * gams[j] + bets[j]
            return c2
        return tok

    def half(b, c):
        qr, ir, gsem, wqsem = bufs[b]
        qro, iro, gsemo, wqsemo = bufs[1 - b]

        @pl.when(c + 1 < _G)
        def _():
            @pl.when(c > 0)
            def _():
                pltpu.make_async_copy(
                    qro, qemb_hbm.at[pl.ds(span0, _C)], wqsemo).wait()
            issue_gathers(c + 1, qro, iro, gsemo)

        drain_gathers(qr, ir, gsem)
        tok0 = pl.multiple_of(span0 + c * _C, _C)
        pltpu.async_copy(qr, qemb_hbm.at[pl.ds(tok0, _C)], wqsem)
        lax.fori_loop(0, _C, make_tok(qr, ir), 0)
        pltpu.sync_copy(ir, x_hbm.at[pl.ds(tok0, _C)])

    issue_gathers(0, qr0, ir0, g0)

    def pair(i, carry):
        half(0, 2 * i)
        half(1, 2 * i + 1)
        return carry

    lax.fori_loop(0, _G // 2, pair, 0)
    pltpu.make_async_copy(qr0, qemb_hbm.at[pl.ds(span0, _C)], wq0).wait()
    pltpu.make_async_copy(qr1, qemb_hbm.at[pl.ds(span0, _C)], wq1).wait()


def kernel(question_ids, responses, question_table, interaction_table,
           ln_gamma, ln_beta):
    qid = question_ids.reshape(_N).astype(jnp.int32)
    resp = responses.reshape(_N).astype(jnp.int32)
    mesh = plsc.VectorSubcoreMesh(core_axis_name="c", subcore_axis_name="s")
    run = pl.kernel(
        _tec_body,
        out_type=(
            jax.ShapeDtypeStruct((_N, _D), jnp.float32),
            jax.ShapeDtypeStruct((_N, _D), jnp.float32),
        ),
        mesh=mesh,
        compiler_params=pltpu.CompilerParams(
            needs_layout_passes=False, use_tc_tiling_on_sc=False),
        scratch_types=[
            pltpu.VMEM((_NT,), jnp.int32),
            pltpu.VMEM((_NT,), jnp.int32),
            pltpu.VMEM((_C, _D), jnp.float32),
            pltpu.VMEM((_C, _D), jnp.float32),
            pltpu.VMEM((_C, _D), jnp.float32),
            pltpu.VMEM((_C, _D), jnp.float32),
            pltpu.VMEM((_D,), jnp.float32),
            pltpu.VMEM((_D,), jnp.float32),
            pltpu.SemaphoreType.DMA,
            pltpu.SemaphoreType.DMA,
            pltpu.SemaphoreType.DMA,
            pltpu.SemaphoreType.DMA,
        ],
    )
    x, qemb = run(qid, resp, question_table, interaction_table,
                  ln_gamma, ln_beta)
    return (x.reshape(_B, _T, _D), qemb.reshape(_B, _T, _D))


# speedup vs baseline: 1.8800x; 1.2979x over previous
import functools

import jax
import jax.numpy as jnp
from jax import lax
from jax.experimental import pallas as pl
from jax.experimental.pallas import tpu as pltpu
from jax.experimental.pallas import tpu_sc as plsc

_NUM_Q = 100000
_D = 64
_B, _T = 1024, 200
_N = _B * _T
_NC, _NS = 2, 16
_NW = _NC * _NS
_NT = _N // _NW
_C = 320
_G = _NT // _C
_MAX_IID = 2 * _NUM_Q
_EPS = 1e-5


def _tec_body(qid_hbm, resp_hbm, qtab_hbm, itab_hbm, gam_hbm, bet_hbm,
              x_hbm, qemb_hbm,
              qspan_v, ispan_v, qr0, ir0, qr1, ir1, gam_v, bet_v,
              g0, g1, wq0, wq1):
    cid = lax.axis_index("c")
    sid = lax.axis_index("s")
    wid = sid * _NC + cid
    span0 = pl.multiple_of(wid * _NT, _NT)
    pltpu.sync_copy(gam_hbm, gam_v)
    pltpu.sync_copy(bet_hbm, bet_v)
    pltpu.sync_copy(qid_hbm.at[pl.ds(span0, _NT)], qspan_v)
    pltpu.sync_copy(resp_hbm.at[pl.ds(span0, _NT)], ispan_v)

    def idx_row(r, c0):
        s = pl.ds(r * 16, 16)
        q = qspan_v[s]
        rr = ispan_v[s]
        rr = jnp.minimum(jnp.maximum(rr, 0), 1)
        iid = q + q + rr
        iid = jnp.where(q == 0, 0, iid)
        ispan_v[s] = jnp.minimum(iid, _MAX_IID)
        return c0

    lax.fori_loop(0, _NT // 16, idx_row, 0)

    bufs = ((qr0, ir0, g0, wq0), (qr1, ir1, g1, wq1))

    def issue_gathers(c, qr, ir, sem):
        off = c * _C
        pltpu.async_copy(qtab_hbm.at[qspan_v.at[pl.ds(off, _C)]], qr, sem)
        pltpu.async_copy(itab_hbm.at[ispan_v.at[pl.ds(off, _C)]], ir, sem)

    def drain_gathers(qr, ir, sem):
        pltpu.make_async_copy(qtab_hbm.at[qspan_v.at[pl.ds(0, _C)]], qr,
                              sem).wait()
        pltpu.make_async_copy(itab_hbm.at[ispan_v.at[pl.ds(0, _C)]], ir,
                              sem).wait()

    inv_d = jnp.float32(1.0 / _D)
    lanes = lax.iota(jnp.int32, 16)
    gams = [gam_v[pl.ds(j * 16, 16)] for j in range(_D // 16)]
    bets = [bet_v[pl.ds(j * 16, 16)] for j in range(_D // 16)]

    def hsum(v):
        for k in (8, 4, 2, 1):
            v = v + v.at[lanes ^ k].get(mode="promise_in_bounds")
        return v

    _U = 4

    def make_tok(qr, ir):
        def tok(i, c2):
            for u in range(_U):
                t = i * _U + u
                v = []
                for j in range(_D // 16):
                    s = pl.ds(j * 16, 16)
                    v.append(qr[t, s] + ir[t, s])
                stot = (v[0] + v[1]) + (v[2] + v[3])
                sstot = (v[0] * v[0] + v[1] * v[1]) + (
                    v[2] * v[2] + v[3] * v[3])
                mv = hsum(stot) * inv_d
                vv = hsum(sstot) * inv_d - mv * mv + jnp.float32(_EPS)
                iy = jnp.int32(0x5F3759DF) - lax.shift_right_logical(
                    plsc.bitcast(vv, jnp.int32), 1)
                y = plsc.bitcast(iy, jnp.float32)
                for _ in range(2):
                    y = y * (jnp.float32(1.5) - jnp.float32(0.5) * vv * y * y)
                nb = mv * y
                for j in range(_D // 16):
                    s = pl.ds(j * 16, 16)
                    ir[t, s] = (v[j] * y - nb) * gams[j] + bets[j]
            return c2
        return tok

    def half(b, c):
        qr, ir, gsem, wqsem = bufs[b]
        qro, iro, gsemo, wqsemo = bufs[1 - b]

        @pl.when(c + 1 < _G)
        def _():
            @pl.when(c > 0)
            def _():
                pltpu.make_async_copy(
                    qro, qemb_hbm.at[pl.ds(span0, _C)], wqsemo).wait()
            issue_gathers(c + 1, qro, iro, gsemo)

        drain_gathers(qr, ir, gsem)
        tok0 = pl.multiple_of(span0 + c * _C, _C)
        pltpu.async_copy(qr, qemb_hbm.at[pl.ds(tok0, _C)], wqsem)
        lax.fori_loop(0, _C // _U, make_tok(qr, ir), 0)
        pltpu.sync_copy(ir, x_hbm.at[pl.ds(tok0, _C)])

    issue_gathers(0, qr0, ir0, g0)

    def pair(i, carry):
        half(0, 2 * i)
        half(1, 2 * i + 1)
        return carry

    lax.fori_loop(0, _G // 2, pair, 0)
    pltpu.make_async_copy(qr0, qemb_hbm.at[pl.ds(span0, _C)], wq0).wait()
    pltpu.make_async_copy(qr1, qemb_hbm.at[pl.ds(span0, _C)], wq1).wait()


def kernel(question_ids, responses, question_table, interaction_table,
           ln_gamma, ln_beta):
    qid = question_ids.reshape(_N).astype(jnp.int32)
    resp = responses.reshape(_N).astype(jnp.int32)
    mesh = plsc.VectorSubcoreMesh(core_axis_name="c", subcore_axis_name="s")
    run = pl.kernel(
        _tec_body,
        out_type=(
            jax.ShapeDtypeStruct((_N, _D), jnp.float32),
            jax.ShapeDtypeStruct((_N, _D), jnp.float32),
        ),
        mesh=mesh,
        compiler_params=pltpu.CompilerParams(
            needs_layout_passes=False, use_tc_tiling_on_sc=False),
        scratch_types=[
            pltpu.VMEM((_NT,), jnp.int32),
            pltpu.VMEM((_NT,), jnp.int32),
            pltpu.VMEM((_C, _D), jnp.float32),
            pltpu.VMEM((_C, _D), jnp.float32),
            pltpu.VMEM((_C, _D), jnp.float32),
            pltpu.VMEM((_C, _D), jnp.float32),
            pltpu.VMEM((_D,), jnp.float32),
            pltpu.VMEM((_D,), jnp.float32),
            pltpu.SemaphoreType.DMA,
            pltpu.SemaphoreType.DMA,
            pltpu.SemaphoreType.DMA,
            pltpu.SemaphoreType.DMA,
        ],
    )
    x, qemb = run(qid, resp, question_table, interaction_table,
                  ln_gamma, ln_beta)
    return (x.reshape(_B, _T, _D), qemb.reshape(_B, _T, _D))
